# baseline (speedup 1.0000x reference)
import functools

import jax
import jax.numpy as jnp
from jax import lax
from jax.experimental import pallas as pl
from jax.experimental.pallas import tpu as pltpu


def _erf(x):
    a1, a2, a3, a4, a5 = (0.254829592, -0.284496736, 1.421413741,
                          -1.453152027, 1.061405429)
    p = 0.3275911
    s = jnp.sign(x)
    z = jnp.abs(x)
    t = 1.0 / (1.0 + p * z)
    poly = t * (a1 + t * (a2 + t * (a3 + t * (a4 + t * a5))))
    y = 1.0 - poly * jnp.exp(-z * z)
    return s * y


def _gelu(x):
    return 0.5 * x * (1.0 + _erf(x * 0.7071067811865476))


def _router_body(x_ref, gw_ref, coef_ref):
    x = x_ref[...]
    gw = gw_ref[...]
    logits = lax.dot_general(x, gw, (((1,), (1,)), ((), ())),
                             preferred_element_type=jnp.float32)
    m = jnp.max(logits, axis=1, keepdims=True)
    ex = jnp.exp(logits - m)
    probs = ex / jnp.sum(ex, axis=1, keepdims=True)
    n, e = probs.shape
    iota = lax.broadcasted_iota(jnp.int32, (n, e), 1)
    m1 = jnp.max(probs, axis=1, keepdims=True)
    a1 = jnp.min(jnp.where(probs == m1, iota, e), axis=1, keepdims=True)
    probs2 = jnp.where(iota == a1, -1.0, probs)
    m2 = jnp.max(probs2, axis=1, keepdims=True)
    a2 = jnp.min(jnp.where(probs2 == m2, iota, e), axis=1, keepdims=True)
    coef = (m1 * (iota == a1).astype(jnp.float32)
            + m2 * (iota == a2).astype(jnp.float32))
    coef_ref[...] = coef


def _ffn_body(x_ref, wup_ref, bup_ref, wdown_ref, bdown_ref, coef_ref,
              out_ref, acc_ref, *, nf):
    e = pl.program_id(1)
    f = pl.program_id(2)

    @pl.when(f == 0)
    def _():
        acc_ref[...] = jnp.zeros_like(acc_ref)

    x = x_ref[...]
    h = lax.dot_general(x, wup_ref[0], (((1,), (0,)), ((), ())),
                        preferred_element_type=jnp.float32)
    h = _gelu(h + bup_ref[0]).astype(wdown_ref.dtype)
    acc_ref[...] += lax.dot_general(h, wdown_ref[0], (((1,), (0,)), ((), ())),
                                    preferred_element_type=jnp.float32)

    @pl.when(f == nf - 1)
    def _():
        coef = coef_ref[...]
        ne = coef.shape[1]
        onehot = (lax.broadcasted_iota(jnp.int32, (ne, 1), 0) == e
                  ).astype(jnp.float32)
        ccol = jnp.dot(coef, onehot, preferred_element_type=jnp.float32)
        contrib = ccol * (acc_ref[...] + bdown_ref[0])

        @pl.when(e == 0)
        def _():
            out_ref[...] = contrib

        @pl.when(e != 0)
        def _():
            out_ref[...] += contrib


def kernel(hidden_states, gate_W, Wup, bup, Wdown, bdown):
    B, S, D = hidden_states.shape
    E, _, DFF = Wup.shape
    N = B * S
    x = hidden_states.reshape(N, D)

    coef = pl.pallas_call(
        _router_body,
        out_shape=jax.ShapeDtypeStruct((N, E), jnp.float32),
    )(x, gate_W)

    TB = min(512, N)
    FT = min(1024, DFF)
    nt, nf = N // TB, DFF // FT

    xb = x.astype(jnp.bfloat16)
    Wupb = Wup.astype(jnp.bfloat16)
    Wdownb = Wdown.astype(jnp.bfloat16)
    bup3 = bup.reshape(E, 1, DFF)
    bdown3 = bdown.reshape(E, 1, D)

    out = pl.pallas_call(
        functools.partial(_ffn_body, nf=nf),
        grid=(nt, E, nf),
        in_specs=[
            pl.BlockSpec((TB, D), lambda i, e, f: (i, 0)),
            pl.BlockSpec((1, D, FT), lambda i, e, f: (e, 0, f)),
            pl.BlockSpec((1, 1, FT), lambda i, e, f: (e, 0, f)),
            pl.BlockSpec((1, FT, D), lambda i, e, f: (e, f, 0)),
            pl.BlockSpec((1, 1, D), lambda i, e, f: (e, 0, 0)),
            pl.BlockSpec((TB, E), lambda i, e, f: (i, 0)),
        ],
        out_specs=pl.BlockSpec((TB, D), lambda i, e, f: (i, 0)),
        out_shape=jax.ShapeDtypeStruct((N, D), jnp.float32),
        scratch_shapes=[pltpu.VMEM((TB, D), jnp.float32)],
    )(xb, Wupb, bup3, Wdownb, bdown3, coef)

    return out.reshape(B, S, D)


# speedup vs baseline: 1.3607x; 1.3607x over previous
import functools

import jax
import jax.numpy as jnp
from jax import lax
from jax.experimental import pallas as pl
from jax.experimental.pallas import tpu as pltpu
from jax.experimental.pallas import tpu_sc as plsc

TGRP = 256
NW = 32
CHT = 8


def _erf(x):
    a1, a2, a3, a4, a5 = (0.254829592, -0.284496736, 1.421413741,
                          -1.453152027, 1.061405429)
    p = 0.3275911
    s = jnp.sign(x)
    z = jnp.abs(x)
    t = 1.0 / (1.0 + p * z)
    poly = t * (a1 + t * (a2 + t * (a3 + t * (a4 + t * a5))))
    return s * (1.0 - poly * jnp.exp(-z * z))


def _gelu(x):
    return 0.5 * x * (1.0 + _erf(x * 0.7071067811865476))


def _router_body(x_ref, gw_ref, w2_ref, a2_ref):
    x = x_ref[...]
    gw = gw_ref[...]
    logits = lax.dot_general(x, gw, (((1,), (1,)), ((), ())),
                             preferred_element_type=jnp.float32)
    m = jnp.max(logits, axis=1, keepdims=True)
    ex = jnp.exp(logits - m)
    probs = ex / jnp.sum(ex, axis=1, keepdims=True)
    n, e = probs.shape
    iota = lax.broadcasted_iota(jnp.int32, (n, e), 1)
    m1 = jnp.max(probs, axis=1, keepdims=True)
    a1 = jnp.min(jnp.where(probs == m1, iota, e), axis=1, keepdims=True)
    probs2 = jnp.where(iota == a1, -1.0, probs)
    m2 = jnp.max(probs2, axis=1, keepdims=True)
    a2 = jnp.min(jnp.where(probs2 == m2, iota, e), axis=1, keepdims=True)
    w2_ref[...] = jnp.concatenate([m1, m2], axis=1)
    a2_ref[...] = jnp.concatenate([a1, a2], axis=1)


def _dispatch_body(a2_ref, slot_ref, blk_ref, *, n_tok, n_e, nb_pad):
    a2 = a2_ref[...]
    iota_e = lax.broadcasted_iota(jnp.int32, (n_tok, n_e), 1)
    oh0 = (iota_e == a2[:, 0:1]).astype(jnp.float32)
    oh1 = (iota_e == a2[:, 1:2]).astype(jnp.float32)
    ohs = oh0 + oh1

    ones_col = jnp.ones((n_tok, 1), jnp.float32)
    cnt_col = lax.dot_general(ohs, ones_col, (((0,), (0,)), ((), ())),
                              preferred_element_type=jnp.float32)
    cnt_i = cnt_col.astype(jnp.int32)
    pc_i = ((cnt_i + (TGRP - 1)) >> 8) << 8
    pc = pc_i.astype(jnp.float32)
    mle = (lax.broadcasted_iota(jnp.int32, (n_e, n_e), 1)
           <= lax.broadcasted_iota(jnp.int32, (n_e, n_e), 0)
           ).astype(jnp.float32)
    base_incl = jnp.dot(mle, pc, preferred_element_type=jnp.float32)
    base_excl = base_incl - pc

    nblk = n_tok // 128
    oh3 = ohs.reshape(nblk, 128, n_e)
    bsums = jnp.sum(oh3, axis=1)
    lt_b = (lax.broadcasted_iota(jnp.int32, (nblk, nblk), 1)
            < lax.broadcasted_iota(jnp.int32, (nblk, nblk), 0)
            ).astype(jnp.float32)
    pfx = jnp.dot(lt_b, bsums, preferred_element_type=jnp.float32)
    lt128 = (lax.broadcasted_iota(jnp.int32, (128, 128), 1)
             < lax.broadcasted_iota(jnp.int32, (128, 128), 0)
             ).astype(jnp.float32)

    look0 = lax.dot_general(oh0, base_excl, (((1,), (0,)), ((), ())),
                            preferred_element_type=jnp.float32)
    look1 = lax.dot_general(oh1, base_excl, (((1,), (0,)), ((), ())),
                            preferred_element_type=jnp.float32)

    slots = []
    for b in range(nblk):
        c_b = (jnp.dot(lt128, oh3[b], preferred_element_type=jnp.float32)
               + pfx[b:b + 1, :])
        r0 = jnp.sum(oh0[b * 128:(b + 1) * 128] * c_b, axis=1, keepdims=True)
        r1 = jnp.sum(oh1[b * 128:(b + 1) * 128] * c_b, axis=1, keepdims=True)
        s0 = look0[b * 128:(b + 1) * 128] + r0
        s1 = look1[b * 128:(b + 1) * 128] + r1
        slots.append(jnp.concatenate([s0, s1], axis=1))
    slot2 = jnp.concatenate(slots, axis=0).astype(jnp.int32)
    slot_ref[...] = slot2

    ntf = (lax.broadcasted_iota(jnp.int32, (1, nb_pad), 1) * TGRP
           ).astype(jnp.float32)
    xg = jnp.dot(jnp.ones((n_e, 1), jnp.float32), ntf,
                 preferred_element_type=jnp.float32)
    yg = jnp.dot(base_incl, jnp.ones((1, nb_pad), jnp.float32),
                 preferred_element_type=jnp.float32)
    be = jnp.sum((xg >= yg).astype(jnp.float32), axis=0, keepdims=True)
    blk_ref[...] = jnp.minimum(be, float(n_e - 1)).astype(jnp.int32)


def _sc_scatter_body(x_hbm, se_hbm, so_hbm, out_hbm, xbuf, se_v, so_v):
    wid = lax.axis_index("s") * 2 + lax.axis_index("c")
    tok_per_w = x_hbm.shape[0] // NW
    nch = tok_per_w // CHT
    base = wid * tok_per_w
    pltpu.sync_copy(se_hbm.at[wid], se_v)
    pltpu.sync_copy(so_hbm.at[wid], so_v)
    for c in range(nch):
        pltpu.sync_copy(x_hbm.at[pl.ds(base + c * CHT, CHT)], xbuf)
        pltpu.sync_copy(xbuf, out_hbm.at[se_v.at[c]])
        pltpu.sync_copy(xbuf, out_hbm.at[so_v.at[c]])


def _sc_gather_body(y_hbm, pe_hbm, po_hbm, z0_hbm, z1_hbm,
                    buf0, buf1, pe_v, po_v, sem0, sem1):
    wid = lax.axis_index("s") * 2 + lax.axis_index("c")
    tok_per_w = z0_hbm.shape[0] // NW
    nch = tok_per_w // CHT
    base = wid * tok_per_w
    pltpu.sync_copy(pe_hbm.at[wid], pe_v)
    pltpu.sync_copy(po_hbm.at[wid], po_v)
    for c in range(nch):
        cp0 = pltpu.async_copy(y_hbm.at[pe_v.at[c]], buf0, sem0)
        cp1 = pltpu.async_copy(y_hbm.at[po_v.at[c]], buf1, sem1)
        cp0.wait()
        cp1.wait()
        pltpu.sync_copy(buf0, z0_hbm.at[pl.ds(base + c * CHT, CHT)])
        pltpu.sync_copy(buf1, z1_hbm.at[pl.ds(base + c * CHT, CHT)])


def _gemm_body(be_ref, x_ref, wup_ref, bup_ref, wdown_ref, bdown_ref,
               out_ref):
    x = x_ref[...]
    h = lax.dot_general(x, wup_ref[0], (((1,), (0,)), ((), ())),
                        preferred_element_type=jnp.float32)
    h = _gelu(h + bup_ref[0]).astype(jnp.bfloat16)
    y = lax.dot_general(h, wdown_ref[0], (((1,), (0,)), ((), ())),
                        preferred_element_type=jnp.float32)
    out_ref[...] = y + bdown_ref[0]


def _combine_body(z0_ref, z1_ref, w0_ref, w1_ref, out_ref):
    out_ref[...] = (w0_ref[...] * z0_ref[...] + w1_ref[...] * z1_ref[...])


def kernel(hidden_states, gate_W, Wup, bup, Wdown, bdown):
    B, S, D = hidden_states.shape
    E, _, DFF = Wup.shape
    N = B * S
    N2 = 2 * N
    NB = N2 // TGRP + E
    NB_PAD = ((NB + 15) // 16) * 16
    P = NB * TGRP
    x = hidden_states.reshape(N, D)

    TBR = min(512, N)
    w2, a2 = pl.pallas_call(
        _router_body,
        grid=(N // TBR,),
        in_specs=[pl.BlockSpec((TBR, D), lambda i: (i, 0)),
                  pl.BlockSpec((E, D), lambda i: (0, 0))],
        out_specs=[pl.BlockSpec((TBR, 2), lambda i: (i, 0)),
                   pl.BlockSpec((TBR, 2), lambda i: (i, 0))],
        out_shape=[jax.ShapeDtypeStruct((N, 2), jnp.float32),
                   jax.ShapeDtypeStruct((N, 2), jnp.int32)],
    )(x, gate_W)

    slot2, blk_e2 = pl.pallas_call(
        functools.partial(_dispatch_body, n_tok=N, n_e=E, nb_pad=NB_PAD),
        out_shape=[jax.ShapeDtypeStruct((N, 2), jnp.int32),
                   jax.ShapeDtypeStruct((1, NB_PAD), jnp.int32)],
    )(a2)
    blk_e = blk_e2[0, :NB]

    xb = x.astype(jnp.bfloat16)
    xi = lax.bitcast_convert_type(xb.reshape(N, D // 2, 2), jnp.int32)
    se = slot2[:, 0].reshape(NW, N // NW // CHT, CHT)
    so = slot2[:, 1].reshape(NW, N // NW // CHT, CHT)
    mesh = plsc.VectorSubcoreMesh(core_axis_name="c", subcore_axis_name="s")
    xs_i = pl.kernel(
        _sc_scatter_body,
        mesh=mesh,
        out_type=jax.ShapeDtypeStruct((P, D // 2), jnp.int32),
        scratch_types=[
            pltpu.VMEM((CHT, D // 2), jnp.int32),
            pltpu.VMEM((N // NW // CHT, CHT), jnp.int32),
            pltpu.VMEM((N // NW // CHT, CHT), jnp.int32),
        ],
    )(xi, se, so)
    xs = lax.bitcast_convert_type(xs_i, jnp.bfloat16).reshape(P, D)

    Wupb = Wup.astype(jnp.bfloat16)
    Wdownb = Wdown.astype(jnp.bfloat16)
    bup3 = bup.reshape(E, 1, DFF)
    bdown3 = bdown.reshape(E, 1, D)
    grid_spec = pltpu.PrefetchScalarGridSpec(
        num_scalar_prefetch=1,
        grid=(NB,),
        in_specs=[
            pl.BlockSpec((TGRP, D), lambda i, be: (i, 0)),
            pl.BlockSpec((1, D, DFF), lambda i, be: (be[i], 0, 0)),
            pl.BlockSpec((1, 1, DFF), lambda i, be: (be[i], 0, 0)),
            pl.BlockSpec((1, DFF, D), lambda i, be: (be[i], 0, 0)),
            pl.BlockSpec((1, 1, D), lambda i, be: (be[i], 0, 0)),
        ],
        out_specs=pl.BlockSpec((TGRP, D), lambda i, be: (i, 0)),
    )
    ys = pl.pallas_call(
        _gemm_body,
        grid_spec=grid_spec,
        out_shape=jax.ShapeDtypeStruct((P, D), jnp.float32),
    )(blk_e, xs, Wupb, bup3, Wdownb, bdown3)

    z0, z1 = pl.kernel(
        _sc_gather_body,
        mesh=mesh,
        out_type=[jax.ShapeDtypeStruct((N, D), jnp.float32),
                  jax.ShapeDtypeStruct((N, D), jnp.float32)],
        scratch_types=[
            pltpu.VMEM((CHT, D), jnp.float32),
            pltpu.VMEM((CHT, D), jnp.float32),
            pltpu.VMEM((N // NW // CHT, CHT), jnp.int32),
            pltpu.VMEM((N // NW // CHT, CHT), jnp.int32),
            pltpu.SemaphoreType.DMA,
            pltpu.SemaphoreType.DMA,
        ],
    )(ys, se, so)

    TBC = min(1024, N)
    out = pl.pallas_call(
        _combine_body,
        grid=(N // TBC,),
        in_specs=[pl.BlockSpec((TBC, D), lambda i: (i, 0)),
                  pl.BlockSpec((TBC, D), lambda i: (i, 0)),
                  pl.BlockSpec((TBC, 1), lambda i: (i, 0)),
                  pl.BlockSpec((TBC, 1), lambda i: (i, 0))],
        out_specs=pl.BlockSpec((TBC, D), lambda i: (i, 0)),
        out_shape=jax.ShapeDtypeStruct((N, D), jnp.float32),
    )(z0, z1, w2[:, 0:1], w2[:, 1:2])

    return out.reshape(B, S, D)


# speedup vs baseline: 2.6048x; 1.9144x over previous
import functools

import jax
import jax.numpy as jnp
from jax import lax
from jax.experimental import pallas as pl
from jax.experimental.pallas import tpu as pltpu
from jax.experimental.pallas import tpu_sc as plsc

TGRP = 256
NW = 32
CHT = 8


def _erf(x):
    a1, a2, a3, a4, a5 = (0.254829592, -0.284496736, 1.421413741,
                          -1.453152027, 1.061405429)
    p = 0.3275911
    s = jnp.sign(x)
    z = jnp.abs(x)
    t = 1.0 / (1.0 + p * z)
    poly = t * (a1 + t * (a2 + t * (a3 + t * (a4 + t * a5))))
    return s * (1.0 - poly * jnp.exp(-z * z))


def _gelu(x):
    return 0.5 * x * (1.0 + _erf(x * 0.7071067811865476))


def _router_body(x_ref, gw_ref, w2_ref, a2_ref):
    x = x_ref[...]
    gw = gw_ref[...]
    logits = lax.dot_general(x, gw, (((1,), (1,)), ((), ())),
                             preferred_element_type=jnp.float32)
    m = jnp.max(logits, axis=1, keepdims=True)
    ex = jnp.exp(logits - m)
    probs = ex / jnp.sum(ex, axis=1, keepdims=True)
    n, e = probs.shape
    iota = lax.broadcasted_iota(jnp.int32, (n, e), 1)
    m1 = jnp.max(probs, axis=1, keepdims=True)
    a1 = jnp.min(jnp.where(probs == m1, iota, e), axis=1, keepdims=True)
    probs2 = jnp.where(iota == a1, -1.0, probs)
    m2 = jnp.max(probs2, axis=1, keepdims=True)
    a2 = jnp.min(jnp.where(probs2 == m2, iota, e), axis=1, keepdims=True)
    w2_ref[...] = jnp.concatenate([m1, m2], axis=1)
    a2_ref[...] = jnp.concatenate([a1, a2], axis=1)


def _dispatch_body(a2_ref, slot_ref, blk_ref, *, n_tok, n_e, nb_pad):
    a2 = a2_ref[...]
    iota_e = lax.broadcasted_iota(jnp.int32, (n_tok, n_e), 1)
    oh0 = (iota_e == a2[:, 0:1]).astype(jnp.float32)
    oh1 = (iota_e == a2[:, 1:2]).astype(jnp.float32)
    ohs = oh0 + oh1

    ones_col = jnp.ones((n_tok, 1), jnp.float32)
    cnt_col = lax.dot_general(ohs, ones_col, (((0,), (0,)), ((), ())),
                              preferred_element_type=jnp.float32)
    cnt_i = cnt_col.astype(jnp.int32)
    pc_i = ((cnt_i + (TGRP - 1)) >> 8) << 8
    pc = pc_i.astype(jnp.float32)
    mle = (lax.broadcasted_iota(jnp.int32, (n_e, n_e), 1)
           <= lax.broadcasted_iota(jnp.int32, (n_e, n_e), 0)
           ).astype(jnp.float32)
    base_incl = jnp.dot(mle, pc, preferred_element_type=jnp.float32)
    base_excl = base_incl - pc

    nblk = n_tok // 128
    oh3 = ohs.reshape(nblk, 128, n_e)
    bsums = jnp.sum(oh3, axis=1)
    lt_b = (lax.broadcasted_iota(jnp.int32, (nblk, nblk), 1)
            < lax.broadcasted_iota(jnp.int32, (nblk, nblk), 0)
            ).astype(jnp.float32)
    pfx = jnp.dot(lt_b, bsums, preferred_element_type=jnp.float32)
    lt128 = (lax.broadcasted_iota(jnp.int32, (128, 128), 1)
             < lax.broadcasted_iota(jnp.int32, (128, 128), 0)
             ).astype(jnp.float32)

    look0 = lax.dot_general(oh0, base_excl, (((1,), (0,)), ((), ())),
                            preferred_element_type=jnp.float32)
    look1 = lax.dot_general(oh1, base_excl, (((1,), (0,)), ((), ())),
                            preferred_element_type=jnp.float32)

    slots = []
    for b in range(nblk):
        c_b = (jnp.dot(lt128, oh3[b], preferred_element_type=jnp.float32)
               + pfx[b:b + 1, :])
        r0 = jnp.sum(oh0[b * 128:(b + 1) * 128] * c_b, axis=1, keepdims=True)
        r1 = jnp.sum(oh1[b * 128:(b + 1) * 128] * c_b, axis=1, keepdims=True)
        s0 = look0[b * 128:(b + 1) * 128] + r0
        s1 = look1[b * 128:(b + 1) * 128] + r1
        slots.append(jnp.concatenate([s0, s1], axis=1))
    slot2 = jnp.concatenate(slots, axis=0).astype(jnp.int32)
    slot_ref[...] = slot2

    ntf = (lax.broadcasted_iota(jnp.int32, (1, nb_pad), 1) * TGRP
           ).astype(jnp.float32)
    xg = jnp.dot(jnp.ones((n_e, 1), jnp.float32), ntf,
                 preferred_element_type=jnp.float32)
    yg = jnp.dot(base_incl, jnp.ones((1, nb_pad), jnp.float32),
                 preferred_element_type=jnp.float32)
    be = jnp.sum((xg >= yg).astype(jnp.float32), axis=0, keepdims=True)
    blk_ref[...] = jnp.minimum(be, float(n_e - 1)).astype(jnp.int32)


def _sc_scatter_body(x_hbm, se_hbm, so_hbm, out_hbm, xbuf, se_v, so_v):
    wid = lax.axis_index("s") * 2 + lax.axis_index("c")
    tok_per_w = x_hbm.shape[0] // NW
    nch = tok_per_w // CHT
    base = wid * tok_per_w
    pltpu.sync_copy(se_hbm.at[wid], se_v)
    pltpu.sync_copy(so_hbm.at[wid], so_v)
    for c in range(nch):
        pltpu.sync_copy(x_hbm.at[pl.ds(base + c * CHT, CHT)], xbuf)
        pltpu.sync_copy(xbuf, out_hbm.at[se_v.at[c]])
        pltpu.sync_copy(xbuf, out_hbm.at[so_v.at[c]])


def _sc_gather_body(y_hbm, pe_hbm, po_hbm, z0_hbm, z1_hbm,
                    buf0, buf1, pe_v, po_v, sem0, sem1):
    wid = lax.axis_index("s") * 2 + lax.axis_index("c")
    tok_per_w = z0_hbm.shape[0] // NW
    nch = tok_per_w // CHT
    base = wid * tok_per_w
    pltpu.sync_copy(pe_hbm.at[wid], pe_v)
    pltpu.sync_copy(po_hbm.at[wid], po_v)
    for c in range(nch):
        cp0 = pltpu.async_copy(y_hbm.at[pe_v.at[c]], buf0, sem0)
        cp1 = pltpu.async_copy(y_hbm.at[po_v.at[c]], buf1, sem1)
        cp0.wait()
        cp1.wait()
        pltpu.sync_copy(buf0, z0_hbm.at[pl.ds(base + c * CHT, CHT)])
        pltpu.sync_copy(buf1, z1_hbm.at[pl.ds(base + c * CHT, CHT)])


def _gelu_tanh(x):
    return 0.5 * x * (1.0 + jnp.tanh(0.7978845608028654
                                     * (x + 0.044715 * x * x * x)))


def _gemm_body(be_ref, x_ref, wup_ref, bup_ref, wdown_ref, bdown_ref,
               out_ref, *, n_fc):
    x = x_ref[...].astype(jnp.bfloat16)
    dff = wup_ref.shape[2]
    fc = dff // n_fc
    acc = None
    for i in range(n_fc):
        sl = slice(i * fc, (i + 1) * fc)
        h = lax.dot_general(x, wup_ref[0, :, sl], (((1,), (0,)), ((), ())),
                            preferred_element_type=jnp.float32)
        g = _gelu_tanh(h + bup_ref[0, :, sl]).astype(jnp.bfloat16)
        d = lax.dot_general(g, wdown_ref[0, sl, :], (((1,), (0,)), ((), ())),
                            preferred_element_type=jnp.float32)
        acc = d if acc is None else acc + d
    out_ref[...] = acc + bdown_ref[0]


def _combine_body(z0_ref, z1_ref, w0_ref, w1_ref, out_ref):
    out_ref[...] = (w0_ref[...] * z0_ref[...] + w1_ref[...] * z1_ref[...])


def kernel(hidden_states, gate_W, Wup, bup, Wdown, bdown):
    B, S, D = hidden_states.shape
    E, _, DFF = Wup.shape
    N = B * S
    N2 = 2 * N
    NB = N2 // TGRP + E
    NB_PAD = ((NB + 15) // 16) * 16
    P = NB * TGRP
    x = hidden_states.reshape(N, D)

    TBR = min(512, N)
    w2, a2 = pl.pallas_call(
        _router_body,
        grid=(N // TBR,),
        in_specs=[pl.BlockSpec((TBR, D), lambda i: (i, 0)),
                  pl.BlockSpec((E, D), lambda i: (0, 0))],
        out_specs=[pl.BlockSpec((TBR, 2), lambda i: (i, 0)),
                   pl.BlockSpec((TBR, 2), lambda i: (i, 0))],
        out_shape=[jax.ShapeDtypeStruct((N, 2), jnp.float32),
                   jax.ShapeDtypeStruct((N, 2), jnp.int32)],
    )(x, gate_W)

    slot2, blk_e2 = pl.pallas_call(
        functools.partial(_dispatch_body, n_tok=N, n_e=E, nb_pad=NB_PAD),
        out_shape=[jax.ShapeDtypeStruct((N, 2), jnp.int32),
                   jax.ShapeDtypeStruct((1, NB_PAD), jnp.int32)],
    )(a2)
    blk_e = blk_e2[0, :NB]

    se = slot2[:, 0].reshape(NW, N // NW // CHT, CHT)
    so = slot2[:, 1].reshape(NW, N // NW // CHT, CHT)
    mesh = plsc.VectorSubcoreMesh(core_axis_name="c", subcore_axis_name="s")
    xs = pl.kernel(
        _sc_scatter_body,
        mesh=mesh,
        out_type=jax.ShapeDtypeStruct((P, D), jnp.float32),
        scratch_types=[
            pltpu.VMEM((CHT, D), jnp.float32),
            pltpu.VMEM((N // NW // CHT, CHT), jnp.int32),
            pltpu.VMEM((N // NW // CHT, CHT), jnp.int32),
        ],
    )(x, se, so)

    Wupb = Wup.astype(jnp.bfloat16)
    Wdownb = Wdown.astype(jnp.bfloat16)
    bup3 = bup.reshape(E, 1, DFF)
    bdown3 = bdown.reshape(E, 1, D)
    grid_spec = pltpu.PrefetchScalarGridSpec(
        num_scalar_prefetch=1,
        grid=(NB,),
        in_specs=[
            pl.BlockSpec((TGRP, D), lambda i, be: (i, 0)),
            pl.BlockSpec((1, D, DFF), lambda i, be: (be[i], 0, 0)),
            pl.BlockSpec((1, 1, DFF), lambda i, be: (be[i], 0, 0)),
            pl.BlockSpec((1, DFF, D), lambda i, be: (be[i], 0, 0)),
            pl.BlockSpec((1, 1, D), lambda i, be: (be[i], 0, 0)),
        ],
        out_specs=pl.BlockSpec((TGRP, D), lambda i, be: (i, 0)),
    )
    ys = pl.pallas_call(
        functools.partial(_gemm_body, n_fc=4),
        grid_spec=grid_spec,
        out_shape=jax.ShapeDtypeStruct((P, D), jnp.float32),
    )(blk_e, xs, Wupb, bup3, Wdownb, bdown3)

    z0, z1 = pl.kernel(
        _sc_gather_body,
        mesh=mesh,
        out_type=[jax.ShapeDtypeStruct((N, D), jnp.float32),
                  jax.ShapeDtypeStruct((N, D), jnp.float32)],
        scratch_types=[
            pltpu.VMEM((CHT, D), jnp.float32),
            pltpu.VMEM((CHT, D), jnp.float32),
            pltpu.VMEM((N // NW // CHT, CHT), jnp.int32),
            pltpu.VMEM((N // NW // CHT, CHT), jnp.int32),
            pltpu.SemaphoreType.DMA,
            pltpu.SemaphoreType.DMA,
        ],
    )(ys, se, so)

    TBC = min(1024, N)
    out = pl.pallas_call(
        _combine_body,
        grid=(N // TBC,),
        in_specs=[pl.BlockSpec((TBC, D), lambda i: (i, 0)),
                  pl.BlockSpec((TBC, D), lambda i: (i, 0)),
                  pl.BlockSpec((TBC, 1), lambda i: (i, 0)),
                  pl.BlockSpec((TBC, 1), lambda i: (i, 0))],
        out_specs=pl.BlockSpec((TBC, D), lambda i: (i, 0)),
        out_shape=jax.ShapeDtypeStruct((N, D), jnp.float32),
    )(z0, z1, w2[:, 0:1], w2[:, 1:2])

    return out.reshape(B, S, D)


# speedup vs baseline: 2.6711x; 1.0254x over previous
import functools

import jax
import jax.numpy as jnp
from jax import lax
from jax.experimental import pallas as pl
from jax.experimental.pallas import tpu as pltpu
from jax.experimental.pallas import tpu_sc as plsc

TGRP = 256
NW = 32
CHT = 16


def _erf(x):
    a1, a2, a3, a4, a5 = (0.254829592, -0.284496736, 1.421413741,
                          -1.453152027, 1.061405429)
    p = 0.3275911
    s = jnp.sign(x)
    z = jnp.abs(x)
    t = 1.0 / (1.0 + p * z)
    poly = t * (a1 + t * (a2 + t * (a3 + t * (a4 + t * a5))))
    return s * (1.0 - poly * jnp.exp(-z * z))


def _gelu(x):
    return 0.5 * x * (1.0 + _erf(x * 0.7071067811865476))


def _router_body(x_ref, gw_ref, w2_ref, a2_ref):
    x = x_ref[...]
    gw = gw_ref[...]
    logits = lax.dot_general(x, gw, (((1,), (1,)), ((), ())),
                             preferred_element_type=jnp.float32)
    m = jnp.max(logits, axis=1, keepdims=True)
    ex = jnp.exp(logits - m)
    probs = ex / jnp.sum(ex, axis=1, keepdims=True)
    n, e = probs.shape
    iota = lax.broadcasted_iota(jnp.int32, (n, e), 1)
    m1 = jnp.max(probs, axis=1, keepdims=True)
    a1 = jnp.min(jnp.where(probs == m1, iota, e), axis=1, keepdims=True)
    probs2 = jnp.where(iota == a1, -1.0, probs)
    m2 = jnp.max(probs2, axis=1, keepdims=True)
    a2 = jnp.min(jnp.where(probs2 == m2, iota, e), axis=1, keepdims=True)
    w2_ref[...] = jnp.concatenate([m1, m2], axis=1)
    a2_ref[...] = jnp.concatenate([a1, a2], axis=1)


def _dispatch_body(a2_ref, slot_ref, blk_ref, *, n_tok, n_e, nb_pad):
    a2 = a2_ref[...]
    iota_e = lax.broadcasted_iota(jnp.int32, (n_tok, n_e), 1)
    oh0 = (iota_e == a2[:, 0:1]).astype(jnp.float32)
    oh1 = (iota_e == a2[:, 1:2]).astype(jnp.float32)
    ohs = oh0 + oh1

    ones_col = jnp.ones((n_tok, 1), jnp.float32)
    cnt_col = lax.dot_general(ohs, ones_col, (((0,), (0,)), ((), ())),
                              preferred_element_type=jnp.float32)
    cnt_i = cnt_col.astype(jnp.int32)
    pc_i = ((cnt_i + (TGRP - 1)) >> 8) << 8
    pc = pc_i.astype(jnp.float32)
    mle = (lax.broadcasted_iota(jnp.int32, (n_e, n_e), 1)
           <= lax.broadcasted_iota(jnp.int32, (n_e, n_e), 0)
           ).astype(jnp.float32)
    base_incl = jnp.dot(mle, pc, preferred_element_type=jnp.float32)
    base_excl = base_incl - pc

    nblk = n_tok // 128
    oh3 = ohs.reshape(nblk, 128, n_e)
    bsums = jnp.sum(oh3, axis=1)
    lt_b = (lax.broadcasted_iota(jnp.int32, (nblk, nblk), 1)
            < lax.broadcasted_iota(jnp.int32, (nblk, nblk), 0)
            ).astype(jnp.float32)
    pfx = jnp.dot(lt_b, bsums, preferred_element_type=jnp.float32)
    lt128 = (lax.broadcasted_iota(jnp.int32, (128, 128), 1)
             < lax.broadcasted_iota(jnp.int32, (128, 128), 0)
             ).astype(jnp.float32)

    look0 = lax.dot_general(oh0, base_excl, (((1,), (0,)), ((), ())),
                            preferred_element_type=jnp.float32)
    look1 = lax.dot_general(oh1, base_excl, (((1,), (0,)), ((), ())),
                            preferred_element_type=jnp.float32)

    slots = []
    for b in range(nblk):
        c_b = (jnp.dot(lt128, oh3[b], preferred_element_type=jnp.float32)
               + pfx[b:b + 1, :])
        r0 = jnp.sum(oh0[b * 128:(b + 1) * 128] * c_b, axis=1, keepdims=True)
        r1 = jnp.sum(oh1[b * 128:(b + 1) * 128] * c_b, axis=1, keepdims=True)
        s0 = look0[b * 128:(b + 1) * 128] + r0
        s1 = look1[b * 128:(b + 1) * 128] + r1
        slots.append(jnp.concatenate([s0, s1], axis=1))
    slot2 = jnp.concatenate(slots, axis=0).astype(jnp.int32)
    slot_ref[...] = slot2

    ntf = (lax.broadcasted_iota(jnp.int32, (1, nb_pad), 1) * TGRP
           ).astype(jnp.float32)
    xg = jnp.dot(jnp.ones((n_e, 1), jnp.float32), ntf,
                 preferred_element_type=jnp.float32)
    yg = jnp.dot(base_incl, jnp.ones((1, nb_pad), jnp.float32),
                 preferred_element_type=jnp.float32)
    be = jnp.sum((xg >= yg).astype(jnp.float32), axis=0, keepdims=True)
    blk_ref[...] = jnp.minimum(be, float(n_e - 1)).astype(jnp.int32)


def _sc_scatter_body(x_hbm, se_hbm, so_hbm, out_hbm,
                     xb0, xb1, se_v, so_v, sem0, sem1):
    wid = lax.axis_index("s") * 2 + lax.axis_index("c")
    tok_per_w = x_hbm.shape[0] // NW
    nch = tok_per_w // CHT
    base = wid * tok_per_w
    pltpu.sync_copy(se_hbm.at[wid], se_v)
    pltpu.sync_copy(so_hbm.at[wid], so_v)
    bufs = (xb0, xb1)
    sems = (sem0, sem1)
    cur = pltpu.async_copy(x_hbm.at[pl.ds(base, CHT)], xb0, sem0)
    for c in range(nch):
        nxt = None
        if c + 1 < nch:
            nxt = pltpu.async_copy(
                x_hbm.at[pl.ds(base + (c + 1) * CHT, CHT)],
                bufs[(c + 1) % 2], sems[(c + 1) % 2])
        cur.wait()
        pltpu.sync_copy(bufs[c % 2], out_hbm.at[se_v.at[c]])
        pltpu.sync_copy(bufs[c % 2], out_hbm.at[so_v.at[c]])
        cur = nxt


def _sc_gather_body(y_hbm, pe_hbm, po_hbm, z0_hbm, z1_hbm,
                    b00, b01, b10, b11, pe_v, po_v, s00, s01, s10, s11):
    wid = lax.axis_index("s") * 2 + lax.axis_index("c")
    tok_per_w = z0_hbm.shape[0] // NW
    nch = tok_per_w // CHT
    base = wid * tok_per_w
    pltpu.sync_copy(pe_hbm.at[wid], pe_v)
    pltpu.sync_copy(po_hbm.at[wid], po_v)
    bufs = ((b00, b01), (b10, b11))
    sems = ((s00, s01), (s10, s11))
    cur = (pltpu.async_copy(y_hbm.at[pe_v.at[0]], b00, s00),
           pltpu.async_copy(y_hbm.at[po_v.at[0]], b01, s01))
    for c in range(nch):
        nxt = None
        if c + 1 < nch:
            b = bufs[(c + 1) % 2]
            s = sems[(c + 1) % 2]
            nxt = (pltpu.async_copy(y_hbm.at[pe_v.at[c + 1]], b[0], s[0]),
                   pltpu.async_copy(y_hbm.at[po_v.at[c + 1]], b[1], s[1]))
        cur[0].wait()
        cur[1].wait()
        pltpu.sync_copy(bufs[c % 2][0], z0_hbm.at[pl.ds(base + c * CHT, CHT)])
        pltpu.sync_copy(bufs[c % 2][1], z1_hbm.at[pl.ds(base + c * CHT, CHT)])
        cur = nxt


def _gelu_tanh(x):
    return 0.5 * x * (1.0 + jnp.tanh(0.7978845608028654
                                     * (x + 0.044715 * x * x * x)))


def _gemm_body(be_ref, x_ref, wup_ref, bup_ref, wdown_ref, bdown_ref,
               out_ref, *, n_fc):
    x = x_ref[...].astype(jnp.bfloat16)
    dff = wup_ref.shape[2]
    fc = dff // n_fc
    acc = None
    for i in range(n_fc):
        sl = slice(i * fc, (i + 1) * fc)
        h = lax.dot_general(x, wup_ref[0, :, sl], (((1,), (0,)), ((), ())),
                            preferred_element_type=jnp.float32)
        g = _gelu_tanh(h + bup_ref[0, :, sl]).astype(jnp.bfloat16)
        d = lax.dot_general(g, wdown_ref[0, sl, :], (((1,), (0,)), ((), ())),
                            preferred_element_type=jnp.float32)
        acc = d if acc is None else acc + d
    out_ref[...] = acc + bdown_ref[0]


def _combine_body(z0_ref, z1_ref, w0_ref, w1_ref, out_ref):
    out_ref[...] = (w0_ref[...] * z0_ref[...] + w1_ref[...] * z1_ref[...])


def kernel(hidden_states, gate_W, Wup, bup, Wdown, bdown):
    B, S, D = hidden_states.shape
    E, _, DFF = Wup.shape
    N = B * S
    N2 = 2 * N
    NB = N2 // TGRP + E
    NB_PAD = ((NB + 15) // 16) * 16
    P = NB * TGRP
    x = hidden_states.reshape(N, D)

    TBR = min(512, N)
    w2, a2 = pl.pallas_call(
        _router_body,
        grid=(N // TBR,),
        in_specs=[pl.BlockSpec((TBR, D), lambda i: (i, 0)),
                  pl.BlockSpec((E, D), lambda i: (0, 0))],
        out_specs=[pl.BlockSpec((TBR, 2), lambda i: (i, 0)),
                   pl.BlockSpec((TBR, 2), lambda i: (i, 0))],
        out_shape=[jax.ShapeDtypeStruct((N, 2), jnp.float32),
                   jax.ShapeDtypeStruct((N, 2), jnp.int32)],
    )(x, gate_W)

    slot2, blk_e2 = pl.pallas_call(
        functools.partial(_dispatch_body, n_tok=N, n_e=E, nb_pad=NB_PAD),
        out_shape=[jax.ShapeDtypeStruct((N, 2), jnp.int32),
                   jax.ShapeDtypeStruct((1, NB_PAD), jnp.int32)],
    )(a2)
    blk_e = blk_e2[0, :NB]

    se = slot2[:, 0].reshape(NW, N // NW // CHT, CHT)
    so = slot2[:, 1].reshape(NW, N // NW // CHT, CHT)
    mesh = plsc.VectorSubcoreMesh(core_axis_name="c", subcore_axis_name="s")
    xs = pl.kernel(
        _sc_scatter_body,
        mesh=mesh,
        out_type=jax.ShapeDtypeStruct((P, D), jnp.float32),
        scratch_types=[
            pltpu.VMEM((CHT, D), jnp.float32),
            pltpu.VMEM((CHT, D), jnp.float32),
            pltpu.VMEM((N // NW // CHT, CHT), jnp.int32),
            pltpu.VMEM((N // NW // CHT, CHT), jnp.int32),
            pltpu.SemaphoreType.DMA,
            pltpu.SemaphoreType.DMA,
        ],
    )(x, se, so)

    Wupb = Wup.astype(jnp.bfloat16)
    Wdownb = Wdown.astype(jnp.bfloat16)
    bup3 = bup.reshape(E, 1, DFF)
    bdown3 = bdown.reshape(E, 1, D)
    grid_spec = pltpu.PrefetchScalarGridSpec(
        num_scalar_prefetch=1,
        grid=(NB,),
        in_specs=[
            pl.BlockSpec((TGRP, D), lambda i, be: (i, 0)),
            pl.BlockSpec((1, D, DFF), lambda i, be: (be[i], 0, 0)),
            pl.BlockSpec((1, 1, DFF), lambda i, be: (be[i], 0, 0)),
            pl.BlockSpec((1, DFF, D), lambda i, be: (be[i], 0, 0)),
            pl.BlockSpec((1, 1, D), lambda i, be: (be[i], 0, 0)),
        ],
        out_specs=pl.BlockSpec((TGRP, D), lambda i, be: (i, 0)),
    )
    ys = pl.pallas_call(
        functools.partial(_gemm_body, n_fc=2),
        grid_spec=grid_spec,
        out_shape=jax.ShapeDtypeStruct((P, D), jnp.float32),
    )(blk_e, xs, Wupb, bup3, Wdownb, bdown3)

    z0, z1 = pl.kernel(
        _sc_gather_body,
        mesh=mesh,
        out_type=[jax.ShapeDtypeStruct((N, D), jnp.float32),
                  jax.ShapeDtypeStruct((N, D), jnp.float32)],
        scratch_types=[
            pltpu.VMEM((CHT, D), jnp.float32),
            pltpu.VMEM((CHT, D), jnp.float32),
            pltpu.VMEM((CHT, D), jnp.float32),
            pltpu.VMEM((CHT, D), jnp.float32),
            pltpu.VMEM((N // NW // CHT, CHT), jnp.int32),
            pltpu.VMEM((N // NW // CHT, CHT), jnp.int32),
            pltpu.SemaphoreType.DMA,
            pltpu.SemaphoreType.DMA,
            pltpu.SemaphoreType.DMA,
            pltpu.SemaphoreType.DMA,
        ],
    )(ys, se, so)

    TBC = min(1024, N)
    out = pl.pallas_call(
        _combine_body,
        grid=(N // TBC,),
        in_specs=[pl.BlockSpec((TBC, D), lambda i: (i, 0)),
                  pl.BlockSpec((TBC, D), lambda i: (i, 0)),
                  pl.BlockSpec((TBC, 1), lambda i: (i, 0)),
                  pl.BlockSpec((TBC, 1), lambda i: (i, 0))],
        out_specs=pl.BlockSpec((TBC, D), lambda i: (i, 0)),
        out_shape=jax.ShapeDtypeStruct((N, D), jnp.float32),
    )(z0, z1, w2[:, 0:1], w2[:, 1:2])

    return out.reshape(B, S, D)
